# baseline (speedup 1.0000x reference)
import functools
import math

import jax
import jax.numpy as jnp
from jax import lax
from jax.experimental import pallas as pl
from jax.experimental.pallas import tpu as pltpu
from jax.experimental.pallas import tpu_sc as plsc

VOCAB = 100000
D_MODEL = 128
BATCH = 4096
HIST = 50
PAIR = 2 * HIST
PAIR_PAD = 112

NC = 2
NS = 16
NW = NC * NS
NSPLIT = 2
NRB = 8
SCALE = math.sqrt(D_MODEL)

_mesh = plsc.VectorSubcoreMesh(core_axis_name="c", subcore_axis_name="s")


def _make_emb(nbatch):
    b_per_w = nbatch // NW
    npair = b_per_w // 2
    assert npair % NRB == 0

    @functools.partial(
        pl.kernel,
        mesh=_mesh,
        out_type=jax.ShapeDtypeStruct((nbatch, HIST, D_MODEL), jnp.float32),
        scratch_types=[
            pltpu.VMEM((npair, PAIR_PAD), jnp.int32),
            pltpu.VMEM((NRB, PAIR, D_MODEL), jnp.float32),
            pltpu.SemaphoreType.DMA,
            pltpu.SemaphoreType.DMA,
        ],
    )
    def _emb_sc(x_hbm, w_hbm, out_hbm, idx_v, rows_v, gsem, osem):
        wid = lax.axis_index("s") * NC + lax.axis_index("c")
        b0 = wid * b_per_w

        pltpu.sync_copy(x_hbm.at[pl.ds(wid * npair, npair)], idx_v)

        def gather_start(p, rb):
            pltpu.async_copy(
                w_hbm.at[idx_v.at[p, pl.ds(0, PAIR)]], rows_v.at[rb], gsem
            )

        def gather_wait(p, rb):
            pltpu.make_async_copy(
                w_hbm.at[idx_v.at[p, pl.ds(0, PAIR)]], rows_v.at[rb], gsem
            ).wait()

        def out_start(p, rb):
            pltpu.async_copy(
                rows_v.at[rb, pl.ds(0, HIST)], out_hbm.at[b0 + 2 * p], osem
            )
            pltpu.async_copy(
                rows_v.at[rb, pl.ds(HIST, HIST)], out_hbm.at[b0 + 2 * p + 1], osem
            )

        def out_wait(p, rb):
            pltpu.make_async_copy(
                rows_v.at[rb, pl.ds(0, HIST)], out_hbm.at[b0 + 2 * p], osem
            ).wait()
            pltpu.make_async_copy(
                rows_v.at[rb, pl.ds(HIST, HIST)], out_hbm.at[b0 + 2 * p + 1], osem
            ).wait()

        def scale_buf(rb):
            rows = rows_v.at[rb]

            @plsc.parallel_loop(0, PAIR, unroll=4)
            def _(k):
                for i in range(D_MODEL // 16):
                    sl = pl.ds(16 * i, 16)
                    rows[k, sl] = rows[k, sl] * SCALE

        for rb in range(NRB):
            gather_start(rb, rb)

        def outer(g, _):
            for rb in range(NRB):
                p = g * NRB + rb
                gather_wait(p, rb)
                scale_buf(rb)
                out_start(p, rb)
                nxt = p + NRB

                @pl.when(nxt < npair)
                def _():
                    out_wait(p, rb)
                    gather_start(nxt, rb)

            return 0

        lax.fori_loop(0, npair // NRB, outer, 0)

        for rb in range(NRB):
            out_wait(npair - NRB + rb, rb)

    return _emb_sc


_emb_half = _make_emb(BATCH // NSPLIT)


def _prep(xh):
    xq = xh.reshape(xh.shape[0] // 2, PAIR)
    return jnp.concatenate([xq, xq[:, : PAIR_PAD - PAIR]], axis=1)


def kernel(x, weight):
    nb = BATCH // NSPLIT
    outs = [
        _emb_half(_prep(x[i * nb : (i + 1) * nb]), weight) for i in range(NSPLIT)
    ]
    return jnp.concatenate(outs, axis=0)


# speedup vs baseline: 1.6273x; 1.6273x over previous
import functools
import math

import jax
import jax.numpy as jnp
from jax import lax
from jax.experimental import pallas as pl
from jax.experimental.pallas import tpu as pltpu
from jax.experimental.pallas import tpu_sc as plsc

VOCAB = 100000
D_MODEL = 128
BATCH = 4096
HIST = 50
PAIR = 2 * HIST
PAIR_PAD = 128

NC = 2
NS = 16
NW = NC * NS
B_PER_W = BATCH // NW
NPAIR = B_PER_W // 2
NRB = 8
SCALE = math.sqrt(D_MODEL)

_mesh = plsc.VectorSubcoreMesh(core_axis_name="c", subcore_axis_name="s")


@functools.partial(
    pl.kernel,
    mesh=_mesh,
    out_type=jax.ShapeDtypeStruct((BATCH, HIST, D_MODEL), jnp.float32),
    scratch_types=[
        pltpu.VMEM((NPAIR, PAIR_PAD), jnp.int32),
        pltpu.VMEM((NRB, PAIR, D_MODEL), jnp.float32),
        pltpu.SemaphoreType.DMA,
        pltpu.SemaphoreType.DMA,
    ],
    compiler_params=pltpu.CompilerParams(use_tc_tiling_on_sc=True),
)
def _emb_sc(x_hbm, w_hbm, out_hbm, idx_v, rows_v, gsem, osem):
    wid = lax.axis_index("s") * NC + lax.axis_index("c")
    b0 = wid * B_PER_W

    pltpu.sync_copy(x_hbm.at[pl.ds(wid * NPAIR, NPAIR)], idx_v)

    def gather_start(p, rb):
        pltpu.async_copy(
            w_hbm.at[idx_v.at[p, pl.ds(0, PAIR)]], rows_v.at[rb], gsem
        )

    def gather_wait(p, rb):
        pltpu.make_async_copy(
            w_hbm.at[idx_v.at[p, pl.ds(0, PAIR)]], rows_v.at[rb], gsem
        ).wait()

    def out_start(p, rb):
        pltpu.async_copy(
            rows_v.at[rb, pl.ds(0, HIST)], out_hbm.at[b0 + 2 * p], osem
        )
        pltpu.async_copy(
            rows_v.at[rb, pl.ds(HIST, HIST)], out_hbm.at[b0 + 2 * p + 1], osem
        )

    def out_wait(p, rb):
        pltpu.make_async_copy(
            rows_v.at[rb, pl.ds(0, HIST)], out_hbm.at[b0 + 2 * p], osem
        ).wait()
        pltpu.make_async_copy(
            rows_v.at[rb, pl.ds(HIST, HIST)], out_hbm.at[b0 + 2 * p + 1], osem
        ).wait()

    def scale_buf(rb):
        rows = rows_v.at[rb]

        @plsc.parallel_loop(0, PAIR, unroll=4)
        def _(k):
            for i in range(D_MODEL // 16):
                sl = pl.ds(16 * i, 16)
                rows[k, sl] = rows[k, sl] * SCALE

    for rb in range(NRB):
        gather_start(rb, rb)

    def outer(g, _):
        for rb in range(NRB):
            p = g * NRB + rb
            gather_wait(p, rb)
            scale_buf(rb)
            out_start(p, rb)
            nxt = p + NRB

            @pl.when(nxt < NPAIR)
            def _():
                out_wait(p, rb)
                gather_start(nxt, rb)

        return 0

    lax.fori_loop(0, NPAIR // NRB, outer, 0)

    for rb in range(NRB):
        out_wait(NPAIR - NRB + rb, rb)


def kernel(x, weight):
    xq = x.reshape(BATCH // 2, PAIR)
    xq = jnp.concatenate([xq, xq[:, : PAIR_PAD - PAIR]], axis=1)
    return _emb_sc(xq, weight)
